# baseline (speedup 1.0000x reference)
import functools

import jax
import jax.numpy as jnp
from jax import lax
from jax.experimental import pallas as pl
from jax.experimental.pallas import tpu as pltpu
from jax.experimental.pallas import tpu_sc as plsc

N_CODES = 8192
DIM = 256
N_TOKENS = 16384
COMMITMENT = 0.25


_TM = 256
_WIN = 4096


def _argmin_body(z2_ref, zt_ref, cb_ref, out_ref):
    z2 = z2_ref[...]
    zt16 = zt_ref[...].reshape(DIM, _TM).astype(jnp.bfloat16)
    m_run = None
    i_run = None
    for start in range(0, N_CODES, _WIN):
        size = min(_WIN, N_CODES - start)
        cb = cb_ref[pl.ds(start, size), :]
        p = lax.dot_general(cb, zt16, (((1,), (0,)), ((), ())),
                            preferred_element_type=jnp.float32)
        d = z2 - 2.0 * p
        m = jnp.min(d, axis=0, keepdims=True)
        iota = lax.broadcasted_iota(jnp.int32, (size, _TM), 0) + start
        bi = jnp.min(jnp.where(d == m, iota, jnp.int32(2**30)),
                     axis=0, keepdims=True)
        if m_run is None:
            m_run, i_run = m, bi
        else:
            carry16 = m_run.astype(jnp.bfloat16).astype(jnp.float32)
            upd = m < carry16
            m_run = jnp.where(upd, m, m_run)
            i_run = jnp.where(upd, bi, i_run)
    out_ref[...] = i_run.reshape(1, 1, _TM)


def _argmin_indices(z2, z3, codebook16):
    q = 1024 // _TM
    grid = (N_TOKENS // _TM,)
    out = pl.pallas_call(
        _argmin_body,
        grid=grid,
        in_specs=[
            pl.BlockSpec((1, _TM), lambda i: (0, i)),
            pl.BlockSpec((1, DIM, _TM), lambda i: (i // q, 0, i % q)),
            pl.BlockSpec((N_CODES, DIM), lambda i: (0, 0)),
        ],
        out_specs=pl.BlockSpec((1, 1, _TM), lambda i: (i, 0, 0)),
        out_shape=jax.ShapeDtypeStruct((N_TOKENS // _TM, 1, _TM), jnp.int32),
    )(z2, z3, codebook16)
    return out.reshape(-1)


_NC, _NS = 2, 16
_NW = _NC * _NS
_BPW = N_TOKENS // _NW
_CHUNK = 128


def _sc_body(cb_hbm, idx2_hbm, zeros_hbm, ones_hbm, zq_hbm, pres_hbm,
             idx2_v, rows_v, zeros_v, ones_v, shared, sem):
    cid = lax.axis_index("c")
    sid = lax.axis_index("s")
    wid = sid * _NC + cid
    base = wid * _BPW
    nrow = _BPW // _CHUNK
    pltpu.sync_copy(idx2_hbm.at[pl.ds(wid * nrow, nrow)], idx2_v)
    for c in range(nrow):
        pltpu.async_copy(cb_hbm.at[idx2_v.at[c]], rows_v, sem).wait()
        pltpu.sync_copy(rows_v, zq_hbm.at[pl.ds(base + c * _CHUNK, _CHUNK)])

    @pl.when(sid == 0)
    def _zero():
        pltpu.sync_copy(zeros_hbm, zeros_v)
        pltpu.sync_copy(zeros_v, shared)

    plsc.subcore_barrier()
    pltpu.sync_copy(ones_hbm, ones_v)
    for c in range(nrow):
        pltpu.sync_copy(ones_v, shared.at[idx2_v.at[c]], add=True)
    plsc.subcore_barrier()

    @pl.when(sid == 0)
    def _flush():
        pltpu.sync_copy(shared, zeros_v)
        pltpu.sync_copy(zeros_v, pres_hbm.at[cid])


def _sc_gather(codebook, indices, zeros, ones):
    mesh = plsc.VectorSubcoreMesh(
        core_axis_name="c", subcore_axis_name="s",
        num_cores=_NC, num_subcores=_NS)
    fn = pl.kernel(
        _sc_body,
        out_type=[
            jax.ShapeDtypeStruct((N_TOKENS, DIM), jnp.float32),
            jax.ShapeDtypeStruct((_NC, N_CODES), jnp.int32),
        ],
        mesh=mesh,
        scratch_types=[
            pltpu.VMEM((_BPW // _CHUNK, _CHUNK), jnp.int32),
            pltpu.VMEM((_CHUNK, DIM), jnp.float32),
            pltpu.VMEM((N_CODES,), jnp.int32),
            pltpu.VMEM((_CHUNK,), jnp.int32),
            pltpu.VMEM_SHARED((N_CODES,), jnp.int32),
            pltpu.SemaphoreType.DMA,
        ],
    )
    return fn(codebook, indices.reshape(N_TOKENS // _CHUNK, _CHUNK),
              zeros, ones)


_TK = 1024


def _epilogue_body(z_ref, q_ref, pres_ref, st_ref, sq_ref, cnt_ref, acc_ref):
    i = pl.program_id(0)
    z = z_ref[...].reshape(DIM, _TK)
    qt = jnp.transpose(q_ref[...])
    dlt = qt - z
    st_ref[...] = (z + dlt).reshape(1, DIM, _TK)

    @pl.when(i == 0)
    def _init():
        acc_ref[0, 0] = 0.0
        ps = jnp.sum(pres_ref[...], axis=0)
        cnt_ref[...] = jnp.sum((ps > 0).astype(jnp.float32)).reshape(1, 1)

    acc_ref[0, 0] += jnp.sum(dlt * dlt)

    @pl.when(i == (N_TOKENS // _TK) - 1)
    def _fin():
        sq_ref[...] = acc_ref[0, 0].reshape(1, 1)


def _epilogue(z3, zq_flat, pres):
    grid = (N_TOKENS // _TK,)
    return pl.pallas_call(
        _epilogue_body,
        grid=grid,
        in_specs=[
            pl.BlockSpec((1, DIM, _TK), lambda i: (i, 0, 0)),
            pl.BlockSpec((_TK, DIM), lambda i: (i, 0)),
            pl.BlockSpec((_NC, N_CODES), lambda i: (0, 0)),
        ],
        out_specs=[
            pl.BlockSpec((1, DIM, _TK), lambda i: (i, 0, 0)),
            pl.BlockSpec((1, 1), lambda i: (0, 0)),
            pl.BlockSpec((1, 1), lambda i: (0, 0)),
        ],
        out_shape=[
            jax.ShapeDtypeStruct((N_TOKENS // _TK, DIM, _TK), jnp.float32),
            jax.ShapeDtypeStruct((1, 1), jnp.float32),
            jax.ShapeDtypeStruct((1, 1), jnp.float32),
        ],
        scratch_shapes=[pltpu.SMEM((1, 1), jnp.float32)],
    )(z3, zq_flat, pres)


def kernel(z, codebook):
    B, C, H, W = z.shape
    z3 = z.reshape(B, C, H * W)
    z2 = (z3 ** 2).sum(axis=1).reshape(1, -1)
    indices = _argmin_indices(z2, z3, codebook.astype(jnp.bfloat16))

    zeros = jnp.zeros((N_CODES,), jnp.int32)
    ones = jnp.ones((_CHUNK,), jnp.int32)
    zq_flat, pres = _sc_gather(codebook, indices, zeros, ones)
    st3, sq, cnt = _epilogue(z3, zq_flat, pres)

    z_q_st = st3.reshape(B, C, H, W)
    m = sq[0, 0] / (B * C * H * W)
    vq_loss = m + COMMITMENT * m
    usage = cnt[0, 0] / N_CODES
    return (z_q_st, vq_loss, usage, indices.reshape(B, H, W))


# speedup vs baseline: 1.1618x; 1.1618x over previous
import functools

import jax
import jax.numpy as jnp
from jax import lax
from jax.experimental import pallas as pl
from jax.experimental.pallas import tpu as pltpu
from jax.experimental.pallas import tpu_sc as plsc

N_CODES = 8192
DIM = 256
N_TOKENS = 16384
COMMITMENT = 0.25


_TM = 512
_WIN = 4096


def _argmin_body(z2_ref, zt_ref, cb_ref, out_ref):
    z2 = z2_ref[...]
    zt16 = zt_ref[...]
    m_run = None
    i_run = None
    for start in range(0, N_CODES, _WIN):
        size = min(_WIN, N_CODES - start)
        cb = cb_ref[pl.ds(start, size), :]
        p = lax.dot_general(cb, zt16, (((1,), (0,)), ((), ())),
                            preferred_element_type=jnp.float32)
        d = z2 - 2.0 * p
        m = jnp.min(d, axis=0, keepdims=True)
        iota = lax.broadcasted_iota(jnp.int32, (size, _TM), 0)
        bi = jnp.min(jnp.where(d == m, iota, jnp.int32(2**30)),
                     axis=0, keepdims=True) + start
        if m_run is None:
            m_run, i_run = m, bi
        else:
            carry16 = m_run.astype(jnp.bfloat16).astype(jnp.float32)
            upd = m < carry16
            m_run = jnp.where(upd, m, m_run)
            i_run = jnp.where(upd, bi, i_run)
    out_ref[...] = i_run.reshape(1, 1, _TM)


def _argmin_indices(z2, zt16, codebook16):
    grid = (N_TOKENS // _TM,)
    out = pl.pallas_call(
        _argmin_body,
        grid=grid,
        in_specs=[
            pl.BlockSpec((1, _TM), lambda i: (0, i)),
            pl.BlockSpec((DIM, _TM), lambda i: (0, i)),
            pl.BlockSpec((N_CODES, DIM), lambda i: (0, 0)),
        ],
        out_specs=pl.BlockSpec((1, 1, _TM), lambda i: (i, 0, 0)),
        out_shape=jax.ShapeDtypeStruct((N_TOKENS // _TM, 1, _TM), jnp.int32),
    )(z2, zt16, codebook16)
    return out.reshape(-1)


_NC, _NS = 2, 16
_NW = _NC * _NS
_BPW = N_TOKENS // _NW
_CHUNK = 128


def _sc_body(cb_hbm, idx2_hbm, zeros_hbm, ones_hbm, zq_hbm, pres_hbm,
             idx2_v, rows_v, zeros_v, ones_v, shared, sem):
    cid = lax.axis_index("c")
    sid = lax.axis_index("s")
    wid = sid * _NC + cid
    base = wid * _BPW
    nrow = _BPW // _CHUNK
    pltpu.sync_copy(idx2_hbm.at[pl.ds(wid * nrow, nrow)], idx2_v)
    for c in range(nrow):
        pltpu.async_copy(cb_hbm.at[idx2_v.at[c]], rows_v, sem).wait()
        pltpu.sync_copy(rows_v, zq_hbm.at[pl.ds(base + c * _CHUNK, _CHUNK)])

    @pl.when(sid == 0)
    def _zero():
        pltpu.sync_copy(zeros_hbm, zeros_v)
        pltpu.sync_copy(zeros_v, shared)

    plsc.subcore_barrier()
    pltpu.sync_copy(ones_hbm, ones_v)
    for c in range(nrow):
        pltpu.sync_copy(ones_v, shared.at[idx2_v.at[c]], add=True)
    plsc.subcore_barrier()

    @pl.when(sid == 0)
    def _flush():
        pltpu.sync_copy(shared, zeros_v)
        pltpu.sync_copy(zeros_v, pres_hbm.at[cid])


def _sc_gather(codebook, indices, zeros, ones):
    mesh = plsc.VectorSubcoreMesh(
        core_axis_name="c", subcore_axis_name="s",
        num_cores=_NC, num_subcores=_NS)
    fn = pl.kernel(
        _sc_body,
        out_type=[
            jax.ShapeDtypeStruct((N_TOKENS, DIM), jnp.float32),
            jax.ShapeDtypeStruct((_NC, N_CODES), jnp.int32),
        ],
        mesh=mesh,
        scratch_types=[
            pltpu.VMEM((_BPW // _CHUNK, _CHUNK), jnp.int32),
            pltpu.VMEM((_CHUNK, DIM), jnp.float32),
            pltpu.VMEM((N_CODES,), jnp.int32),
            pltpu.VMEM((_CHUNK,), jnp.int32),
            pltpu.VMEM_SHARED((N_CODES,), jnp.int32),
            pltpu.SemaphoreType.DMA,
        ],
    )
    return fn(codebook, indices.reshape(N_TOKENS // _CHUNK, _CHUNK),
              zeros, ones)


_TK = 1024


def _epilogue_body(z_ref, q_ref, pres_ref, st_ref, sq_ref, cnt_ref, acc_ref):
    i = pl.program_id(0)
    z = z_ref[...]
    q = q_ref[...]
    dlt = q - z
    st_ref[...] = z + dlt

    @pl.when(i == 0)
    def _init():
        acc_ref[0, 0] = 0.0
        ps = jnp.sum(pres_ref[...], axis=0)
        cnt_ref[...] = jnp.sum((ps > 0).astype(jnp.float32)).reshape(1, 1)

    acc_ref[0, 0] += jnp.sum(dlt * dlt)

    @pl.when(i == (N_TOKENS // _TK) - 1)
    def _fin():
        sq_ref[...] = acc_ref[0, 0].reshape(1, 1)


def _epilogue(z_flat, zq_flat, pres):
    grid = (N_TOKENS // _TK,)
    return pl.pallas_call(
        _epilogue_body,
        grid=grid,
        in_specs=[
            pl.BlockSpec((_TK, DIM), lambda i: (i, 0)),
            pl.BlockSpec((_TK, DIM), lambda i: (i, 0)),
            pl.BlockSpec((_NC, N_CODES), lambda i: (0, 0)),
        ],
        out_specs=[
            pl.BlockSpec((_TK, DIM), lambda i: (i, 0)),
            pl.BlockSpec((1, 1), lambda i: (0, 0)),
            pl.BlockSpec((1, 1), lambda i: (0, 0)),
        ],
        out_shape=[
            jax.ShapeDtypeStruct((N_TOKENS, DIM), jnp.float32),
            jax.ShapeDtypeStruct((1, 1), jnp.float32),
            jax.ShapeDtypeStruct((1, 1), jnp.float32),
        ],
        scratch_shapes=[pltpu.SMEM((1, 1), jnp.float32)],
    )(z_flat, zq_flat, pres)


def kernel(z, codebook):
    B, C, H, W = z.shape
    z_flat = z.transpose(0, 2, 3, 1).reshape(-1, C)
    z2 = (z_flat ** 2).sum(axis=1).reshape(1, -1)
    zt16 = z.transpose(1, 0, 2, 3).reshape(C, B * H * W).astype(jnp.bfloat16)
    indices = _argmin_indices(z2, zt16, codebook.astype(jnp.bfloat16))

    zeros = jnp.zeros((N_CODES,), jnp.int32)
    ones = jnp.ones((_CHUNK,), jnp.int32)
    zq_flat, pres = _sc_gather(codebook, indices, zeros, ones)
    st_flat, sq, cnt = _epilogue(z_flat, zq_flat, pres)

    z_q_st = st_flat.reshape(B, H, W, C).transpose(0, 3, 1, 2)
    m = sq[0, 0] / (B * C * H * W)
    vq_loss = m + COMMITMENT * m
    usage = cnt[0, 0] / N_CODES
    return (z_q_st, vq_loss, usage, indices.reshape(B, H, W))


# speedup vs baseline: 1.2061x; 1.0381x over previous
import functools

import jax
import jax.numpy as jnp
from jax import lax
from jax.experimental import pallas as pl
from jax.experimental.pallas import tpu as pltpu
from jax.experimental.pallas import tpu_sc as plsc

N_CODES = 8192
DIM = 256
N_TOKENS = 16384
COMMITMENT = 0.25


_TM = 1024
_WIN = 4096


def _argmin_body(z2_ref, zt_ref, cb_ref, out_ref):
    z2 = z2_ref[...]
    zt16 = zt_ref[...]
    m_run = None
    i_run = None
    for start in range(0, N_CODES, _WIN):
        size = min(_WIN, N_CODES - start)
        cb = cb_ref[pl.ds(start, size), :]
        p = lax.dot_general(cb, zt16, (((1,), (0,)), ((), ())),
                            preferred_element_type=jnp.float32)
        d = z2 - 2.0 * p
        m = jnp.min(d, axis=0, keepdims=True)
        iota = lax.broadcasted_iota(jnp.int32, (size, _TM), 0)
        bi = jnp.min(jnp.where(d == m, iota, jnp.int32(2**30)),
                     axis=0, keepdims=True) + start
        if m_run is None:
            m_run, i_run = m, bi
        else:
            carry16 = m_run.astype(jnp.bfloat16).astype(jnp.float32)
            upd = m < carry16
            m_run = jnp.where(upd, m, m_run)
            i_run = jnp.where(upd, bi, i_run)
    out_ref[...] = i_run.reshape(1, 1, _TM)


def _argmin_indices(z2, zt16, codebook16):
    grid = (N_TOKENS // _TM,)
    out = pl.pallas_call(
        _argmin_body,
        grid=grid,
        in_specs=[
            pl.BlockSpec((1, _TM), lambda i: (0, i)),
            pl.BlockSpec((DIM, _TM), lambda i: (0, i)),
            pl.BlockSpec((N_CODES, DIM), lambda i: (0, 0)),
        ],
        out_specs=pl.BlockSpec((1, 1, _TM), lambda i: (i, 0, 0)),
        out_shape=jax.ShapeDtypeStruct((N_TOKENS // _TM, 1, _TM), jnp.int32),
    )(z2, zt16, codebook16)
    return out.reshape(-1)


_NC, _NS = 2, 16
_NW = _NC * _NS
_BPW = N_TOKENS // _NW
_CHUNK = 128


def _sc_body(cb_hbm, idx2_hbm, zeros_hbm, ones_hbm, zq_hbm, pres_hbm,
             idx2_v, rows_v, zeros_v, ones_v, shared, sem):
    cid = lax.axis_index("c")
    sid = lax.axis_index("s")
    wid = sid * _NC + cid
    base = wid * _BPW
    nrow = _BPW // _CHUNK
    pltpu.sync_copy(idx2_hbm.at[pl.ds(wid * nrow, nrow)], idx2_v)
    for c in range(nrow):
        pltpu.async_copy(cb_hbm.at[idx2_v.at[c]], rows_v, sem).wait()
        pltpu.sync_copy(rows_v, zq_hbm.at[pl.ds(base + c * _CHUNK, _CHUNK)])

    @pl.when(sid == 0)
    def _zero():
        pltpu.sync_copy(zeros_hbm, zeros_v)
        pltpu.sync_copy(zeros_v, shared)

    plsc.subcore_barrier()
    pltpu.sync_copy(ones_hbm, ones_v)
    for c in range(nrow):
        pltpu.sync_copy(ones_v, shared.at[idx2_v.at[c]], add=True)
    plsc.subcore_barrier()

    @pl.when(sid == 0)
    def _flush():
        pltpu.sync_copy(shared, zeros_v)
        pltpu.sync_copy(zeros_v, pres_hbm.at[cid])


def _sc_gather(codebook, indices, zeros, ones):
    mesh = plsc.VectorSubcoreMesh(
        core_axis_name="c", subcore_axis_name="s",
        num_cores=_NC, num_subcores=_NS)
    fn = pl.kernel(
        _sc_body,
        out_type=[
            jax.ShapeDtypeStruct((N_TOKENS, DIM), jnp.float32),
            jax.ShapeDtypeStruct((_NC, N_CODES), jnp.int32),
        ],
        mesh=mesh,
        scratch_types=[
            pltpu.VMEM((_BPW // _CHUNK, _CHUNK), jnp.int32),
            pltpu.VMEM((_CHUNK, DIM), jnp.float32),
            pltpu.VMEM((N_CODES,), jnp.int32),
            pltpu.VMEM((_CHUNK,), jnp.int32),
            pltpu.VMEM_SHARED((N_CODES,), jnp.int32),
            pltpu.SemaphoreType.DMA,
        ],
    )
    return fn(codebook, indices.reshape(N_TOKENS // _CHUNK, _CHUNK),
              zeros, ones)


_TK = 1024


def _epilogue_body(z_ref, q_ref, pres_ref, st_ref, sq_ref, cnt_ref, acc_ref):
    i = pl.program_id(0)
    z = z_ref[...]
    q = q_ref[...]
    dlt = q - z
    st_ref[...] = z + dlt

    @pl.when(i == 0)
    def _init():
        acc_ref[0, 0] = 0.0
        ps = jnp.sum(pres_ref[...], axis=0)
        cnt_ref[...] = jnp.sum((ps > 0).astype(jnp.float32)).reshape(1, 1)

    acc_ref[0, 0] += jnp.sum(dlt * dlt)

    @pl.when(i == (N_TOKENS // _TK) - 1)
    def _fin():
        sq_ref[...] = acc_ref[0, 0].reshape(1, 1)


def _epilogue(z_flat, zq_flat, pres):
    grid = (N_TOKENS // _TK,)
    return pl.pallas_call(
        _epilogue_body,
        grid=grid,
        in_specs=[
            pl.BlockSpec((_TK, DIM), lambda i: (i, 0)),
            pl.BlockSpec((_TK, DIM), lambda i: (i, 0)),
            pl.BlockSpec((_NC, N_CODES), lambda i: (0, 0)),
        ],
        out_specs=[
            pl.BlockSpec((_TK, DIM), lambda i: (i, 0)),
            pl.BlockSpec((1, 1), lambda i: (0, 0)),
            pl.BlockSpec((1, 1), lambda i: (0, 0)),
        ],
        out_shape=[
            jax.ShapeDtypeStruct((N_TOKENS, DIM), jnp.float32),
            jax.ShapeDtypeStruct((1, 1), jnp.float32),
            jax.ShapeDtypeStruct((1, 1), jnp.float32),
        ],
        scratch_shapes=[pltpu.SMEM((1, 1), jnp.float32)],
    )(z_flat, zq_flat, pres)


def kernel(z, codebook):
    B, C, H, W = z.shape
    z_flat = z.transpose(0, 2, 3, 1).reshape(-1, C)
    z2 = (z_flat ** 2).sum(axis=1).reshape(1, -1)
    zt16 = z.transpose(1, 0, 2, 3).reshape(C, B * H * W).astype(jnp.bfloat16)
    indices = _argmin_indices(z2, zt16, codebook.astype(jnp.bfloat16))

    zeros = jnp.zeros((N_CODES,), jnp.int32)
    ones = jnp.ones((_CHUNK,), jnp.int32)
    zq_flat, pres = _sc_gather(codebook, indices, zeros, ones)
    st_flat, sq, cnt = _epilogue(z_flat, zq_flat, pres)

    z_q_st = st_flat.reshape(B, H, W, C).transpose(0, 3, 1, 2)
    m = sq[0, 0] / (B * C * H * W)
    vq_loss = m + COMMITMENT * m
    usage = cnt[0, 0] / N_CODES
    return (z_q_st, vq_loss, usage, indices.reshape(B, H, W))
